# baseline (speedup 1.0000x reference)
import numpy as np
import jax
import jax.numpy as jnp
from jax.experimental import pallas as pl
from jax.experimental.pallas import tpu as pltpu

_SLOW_FRAMES = 8
_FAST_CHUNKS = 8


def _make_body(idx, T):
    chunk = T // _FAST_CHUNKS

    def _body(frames_ref, slow_ref, fast_ref, sems):
        copies = []
        for k in range(_FAST_CHUNKS):
            copies.append(
                pltpu.make_async_copy(
                    frames_ref.at[:, k * chunk:(k + 1) * chunk],
                    fast_ref.at[:, k * chunk:(k + 1) * chunk],
                    sems.at[k],
                )
            )
        for j, t in enumerate(idx):
            copies.append(
                pltpu.make_async_copy(
                    frames_ref.at[:, t:t + 1],
                    slow_ref.at[:, j:j + 1],
                    sems.at[_FAST_CHUNKS + j],
                )
            )
        for c in copies:
            c.start()
        for c in copies:
            c.wait()

    return _body


def kernel(frames):
    C, T, H, W = frames.shape
    idx = [int(v) for v in np.linspace(0.0, float(T - 1), _SLOW_FRAMES).astype(np.int32)]

    slow, fast = pl.pallas_call(
        _make_body(idx, T),
        in_specs=[pl.BlockSpec(memory_space=pltpu.MemorySpace.HBM)],
        out_specs=(
            pl.BlockSpec(memory_space=pltpu.MemorySpace.HBM),
            pl.BlockSpec(memory_space=pltpu.MemorySpace.HBM),
        ),
        out_shape=(
            jax.ShapeDtypeStruct((C, _SLOW_FRAMES, H, W), frames.dtype),
            jax.ShapeDtypeStruct((C, T, H, W), frames.dtype),
        ),
        scratch_shapes=[pltpu.SemaphoreType.DMA((_FAST_CHUNKS + _SLOW_FRAMES,))],
    )(frames)
    return (slow, fast)


# speedup vs baseline: 40.0141x; 40.0141x over previous
import numpy as np
import jax
import jax.numpy as jnp
from jax.experimental import pallas as pl
from jax.experimental.pallas import tpu as pltpu

_SLOW_FRAMES = 8


def _make_body(idx):
    def _body(frames_ref, slow_ref, vmem, in_sems, out_sems):
        n = len(idx)
        ins = [
            pltpu.make_async_copy(
                frames_ref.at[:, t:t + 1], vmem.at[:, j:j + 1], in_sems.at[j]
            )
            for j, t in enumerate(idx)
        ]
        outs = [
            pltpu.make_async_copy(
                vmem.at[:, j:j + 1], slow_ref.at[:, j:j + 1], out_sems.at[j]
            )
            for j in range(n)
        ]
        for c in ins:
            c.start()
        for j in range(n):
            ins[j].wait()
            outs[j].start()
        for c in outs:
            c.wait()

    return _body


def kernel(frames):
    C, T, H, W = frames.shape
    idx = [int(v) for v in np.linspace(0.0, float(T - 1), _SLOW_FRAMES).astype(np.int32)]

    slow = pl.pallas_call(
        _make_body(idx),
        in_specs=[pl.BlockSpec(memory_space=pltpu.MemorySpace.HBM)],
        out_specs=pl.BlockSpec(memory_space=pltpu.MemorySpace.HBM),
        out_shape=jax.ShapeDtypeStruct((C, _SLOW_FRAMES, H, W), frames.dtype),
        scratch_shapes=[
            pltpu.VMEM((C, _SLOW_FRAMES, H, W), frames.dtype),
            pltpu.SemaphoreType.DMA((_SLOW_FRAMES,)),
            pltpu.SemaphoreType.DMA((_SLOW_FRAMES,)),
        ],
    )(frames)
    return (slow, frames)
